# Initial kernel scaffold; baseline (speedup 1.0000x reference)
import jax
import jax.numpy as jnp
from jax.experimental import pallas as pl


def kernel(hidden_states, expert_affinities, expert_index, gate_up_proj, down_proj):
    raise NotImplementedError("write your pallas kernel here")


# speedup vs baseline: 1.0538x; 1.0538x over previous
import functools

import jax
import jax.numpy as jnp
from jax.experimental import pallas as pl
from jax.experimental.pallas import tpu as pltpu


def _moe_body(x_ref, aff_ref, idx_ref, gate_ref, up_ref, down_ref, out_ref,
              w_ref, *, top_k):
    e = pl.program_id(0)
    i = pl.program_id(1)

    @pl.when((e == 0) & (i == 0))
    def _init():
        t, num_e = w_ref.shape
        idx = idx_ref[...]
        erange = jax.lax.broadcasted_iota(jnp.int32, (t, num_e), 1)
        mask = jnp.zeros((t, num_e), jnp.float32)
        for k in range(top_k):
            mask = mask + (idx[:, k][:, None] == erange).astype(jnp.float32)
        w = jnp.where(mask == 0.0, 0.0, aff_ref[...])
        denom = jnp.maximum(jnp.sum(jnp.abs(w), axis=1, keepdims=True), 1e-12)
        w_ref[...] = w / denom
        out_ref[...] = jnp.zeros_like(out_ref)

    x = x_ref[...].astype(jnp.bfloat16)
    gate = jnp.dot(x, gate_ref[0].astype(jnp.bfloat16),
                   preferred_element_type=jnp.float32)
    up = jnp.dot(x, up_ref[0].astype(jnp.bfloat16),
                 preferred_element_type=jnp.float32)
    inter = (gate * jax.lax.logistic(gate) * up).astype(jnp.bfloat16)
    part = jnp.dot(inter, down_ref[0].astype(jnp.bfloat16),
                   preferred_element_type=jnp.float32)
    w_full = w_ref[...]
    col = jax.lax.broadcasted_iota(jnp.int32, w_full.shape, 1)
    we = jnp.sum(jnp.where(col == e, w_full, 0.0), axis=1, keepdims=True)
    out_ref[...] += part * we


def kernel(hidden_states, expert_affinities, expert_index, gate_up_proj,
           down_proj):
    t, h = hidden_states.shape
    num_e = expert_affinities.shape[1]
    top_k = expert_index.shape[1]
    inter_dim = down_proj.shape[1]
    tile_i = min(512, inter_dim)
    ni = inter_dim // tile_i
    expert_index = expert_index.astype(jnp.int32)

    body = functools.partial(_moe_body, top_k=top_k)
    return pl.pallas_call(
        body,
        grid=(num_e, ni),
        in_specs=[
            pl.BlockSpec((t, h), lambda e, i: (0, 0)),
            pl.BlockSpec((t, num_e), lambda e, i: (0, 0)),
            pl.BlockSpec((t, top_k), lambda e, i: (0, 0)),
            pl.BlockSpec((1, h, tile_i), lambda e, i: (e, 0, i)),
            pl.BlockSpec((1, h, tile_i), lambda e, i: (e, 0, ni + i)),
            pl.BlockSpec((1, tile_i, h), lambda e, i: (e, i, 0)),
        ],
        out_specs=pl.BlockSpec((t, h), lambda e, i: (0, 0)),
        out_shape=jax.ShapeDtypeStruct((t, h), jnp.float32),
        scratch_shapes=[pltpu.VMEM((t, num_e), jnp.float32)],
    )(hidden_states, expert_affinities, expert_index, gate_up_proj,
      gate_up_proj, down_proj)


# speedup vs baseline: 1.0632x; 1.0090x over previous
import functools

import jax
import jax.numpy as jnp
from jax.experimental import pallas as pl
from jax.experimental.pallas import tpu as pltpu


def _moe_body(x_ref, aff_ref, idx_ref, gate_ref, up_ref, down_ref, out_ref,
              w_ref, *, top_k):
    e = pl.program_id(0)
    i = pl.program_id(1)

    @pl.when((e == 0) & (i == 0))
    def _init():
        t, num_e = w_ref.shape
        idx = idx_ref[...]
        erange = jax.lax.broadcasted_iota(jnp.int32, (t, num_e), 1)
        mask = jnp.zeros((t, num_e), jnp.float32)
        for k in range(top_k):
            mask = mask + (idx[:, k][:, None] == erange).astype(jnp.float32)
        w = jnp.where(mask == 0.0, 0.0, aff_ref[...])
        denom = jnp.maximum(jnp.sum(jnp.abs(w), axis=1, keepdims=True), 1e-12)
        w_ref[...] = w / denom
        out_ref[...] = jnp.zeros_like(out_ref)

    x = x_ref[...].astype(jnp.bfloat16)
    gate = jnp.dot(x, gate_ref[0].astype(jnp.bfloat16),
                   preferred_element_type=jnp.float32)
    up = jnp.dot(x, up_ref[0].astype(jnp.bfloat16),
                 preferred_element_type=jnp.float32)
    inter = (gate * jax.lax.logistic(gate) * up).astype(jnp.bfloat16)
    part = jnp.dot(inter, down_ref[0].astype(jnp.bfloat16),
                   preferred_element_type=jnp.float32)
    w_full = w_ref[...]
    col = jax.lax.broadcasted_iota(jnp.int32, w_full.shape, 1)
    we = jnp.sum(jnp.where(col == e, w_full, 0.0), axis=1, keepdims=True)
    out_ref[...] += part * we


def kernel(hidden_states, expert_affinities, expert_index, gate_up_proj,
           down_proj):
    t, h = hidden_states.shape
    num_e = expert_affinities.shape[1]
    top_k = expert_index.shape[1]
    inter_dim = down_proj.shape[1]
    tile_i = min(1024, inter_dim)
    ni = inter_dim // tile_i
    expert_index = expert_index.astype(jnp.int32)

    body = functools.partial(_moe_body, top_k=top_k)
    return pl.pallas_call(
        body,
        grid=(num_e, ni),
        in_specs=[
            pl.BlockSpec((t, h), lambda e, i: (0, 0)),
            pl.BlockSpec((t, num_e), lambda e, i: (0, 0)),
            pl.BlockSpec((t, top_k), lambda e, i: (0, 0)),
            pl.BlockSpec((1, h, tile_i), lambda e, i: (e, 0, i)),
            pl.BlockSpec((1, h, tile_i), lambda e, i: (e, 0, ni + i)),
            pl.BlockSpec((1, tile_i, h), lambda e, i: (e, i, 0)),
        ],
        out_specs=pl.BlockSpec((t, h), lambda e, i: (0, 0)),
        out_shape=jax.ShapeDtypeStruct((t, h), jnp.float32),
        scratch_shapes=[pltpu.VMEM((t, num_e), jnp.float32)],
    )(hidden_states, expert_affinities, expert_index, gate_up_proj,
      gate_up_proj, down_proj)
